# baseline (speedup 1.0000x reference)
import jax
import jax.numpy as jnp
from jax.experimental import pallas as pl
from jax.experimental.pallas import tpu as pltpu

_N = 8192
_BI = 512
_BJ = 2048
_NI = _N // _BI
_NJ = _N // _BJ


def _chamfer_block_kernel(p_ref, t_ref, out_ref, rowacc_ref, colacc_ref,
                          sum_ref):
    i = pl.program_id(0)
    j = pl.program_id(1)

    p = p_ref[...]
    t = t_ref[...]

    bf = jnp.bfloat16
    f32 = jnp.float32
    px, py, pz = p[:, 0:1], p[:, 1:2], p[:, 2:3]
    tx, ty, tz = t[0:1, :], t[1:2, :], t[2:3, :]
    pn = px * px + py * py + pz * pz
    tn = tx * tx + ty * ty + tz * tz
    pnh = pn.astype(bf)
    pnl = (pn - pnh.astype(f32)).astype(bf)
    tnh = tn.astype(bf)
    tnl = (tn - tnh.astype(f32)).astype(bf)
    ones_p = jnp.ones((p.shape[0], 1), bf)
    ones_t = jnp.ones((1, t.shape[1]), bf)
    a = jnp.concatenate(
        [(-2.0 * px.astype(bf).astype(f32)).astype(bf),
         (-2.0 * py.astype(bf).astype(f32)).astype(bf),
         (-2.0 * pz.astype(bf).astype(f32)).astype(bf),
         pnh, pnl, ones_p, ones_p], axis=1)
    b = jnp.concatenate(
        [tx.astype(bf), ty.astype(bf), tz.astype(bf),
         ones_t, ones_t, tnh, tnl], axis=0)
    d = jax.lax.dot_general(a, b, (((1,), (0,)), ((), ())),
                            preferred_element_type=f32)

    @pl.when(j == 0)
    def _():
        rowacc_ref[...] = d

    @pl.when(j > 0)
    def _():
        rowacc_ref[...] = jnp.minimum(rowacc_ref[...], d)

    @pl.when(i == 0)
    def _():
        colacc_ref[:, pl.ds(j * _BJ, _BJ)] = d

    @pl.when(i > 0)
    def _():
        colacc_ref[:, pl.ds(j * _BJ, _BJ)] = jnp.minimum(
            colacc_ref[:, pl.ds(j * _BJ, _BJ)], d)

    @pl.when(jnp.logical_and(i == 0, j == 0))
    def _():
        sum_ref[0] = 0.0

    @pl.when(j == _NJ - 1)
    def _():
        sum_ref[0] += jnp.sum(jnp.min(rowacc_ref[...], axis=1))

    @pl.when(jnp.logical_and(i == _NI - 1, j == _NJ - 1))
    def _():
        total = sum_ref[0] + jnp.sum(jnp.min(colacc_ref[...], axis=0))
        out_ref[...] = jnp.full((1, 1), total / (2.0 * _N), jnp.float32)


def _chamfer(pred, target_t, interpret=False):
    return pl.pallas_call(
        _chamfer_block_kernel,
        grid=(_NI, _NJ),
        in_specs=[
            pl.BlockSpec((_BI, 3), lambda i, j: (i, 0)),
            pl.BlockSpec((3, _BJ), lambda i, j: (0, j)),
        ],
        out_specs=pl.BlockSpec((1, 1), lambda i, j: (0, 0)),
        out_shape=jax.ShapeDtypeStruct((1, 1), jnp.float32),
        scratch_shapes=[
            pltpu.VMEM((_BI, _BJ), jnp.float32),
            pltpu.VMEM((_BI, _N), jnp.float32),
            pltpu.SMEM((1,), jnp.float32),
        ],
        interpret=interpret,
    )(pred, target_t)


@jax.jit
def kernel(pred_positions, target_positions):
    out = _chamfer(pred_positions, target_positions.T)
    return out[0, 0]


# speedup vs baseline: 1.4501x; 1.4501x over previous
import jax
import jax.numpy as jnp
from jax import lax
from jax.experimental import pallas as pl
from jax.experimental.pallas import tpu as pltpu

_N = 8192
_BI = 512
_BJ = 2048
_NI = _N // _BI
_NJ = _N // _BJ
_NSTEP = _NI * _NJ


def _make_block(p, t):
    bf = jnp.bfloat16
    f32 = jnp.float32
    px, py, pz = p[:, 0:1], p[:, 1:2], p[:, 2:3]
    tx, ty, tz = t[0:1, :], t[1:2, :], t[2:3, :]
    pn = px * px + py * py + pz * pz
    tn = tx * tx + ty * ty + tz * tz
    pnh = pn.astype(bf)
    pnl = (pn - pnh.astype(f32)).astype(bf)
    tnh = tn.astype(bf)
    tnl = (tn - tnh.astype(f32)).astype(bf)
    ones_p = jnp.ones((p.shape[0], 1), bf)
    ones_t = jnp.ones((1, t.shape[1]), bf)
    a = jnp.concatenate(
        [(-2.0 * px.astype(bf).astype(f32)).astype(bf),
         (-2.0 * py.astype(bf).astype(f32)).astype(bf),
         (-2.0 * pz.astype(bf).astype(f32)).astype(bf),
         pnh, pnl, ones_p, ones_p], axis=1)
    b = jnp.concatenate(
        [tx.astype(bf), ty.astype(bf), tz.astype(bf),
         ones_t, ones_t, tnh, tnl], axis=0)
    return jax.lax.dot_general(a, b, (((1,), (0,)), ((), ())),
                               preferred_element_type=jnp.float32)


def _chamfer_block_kernel(p_ref, t_ref, out_ref, dbufa_ref, dbufb_ref,
                          rowacc_ref, colacc_ref, sum_ref):
    s = pl.program_id(0)

    @pl.when(s == 0)
    def _():
        sum_ref[0] = 0.0

    def _reduce(dref):
        d = dref[...]
        rowm = jnp.min(d, axis=1, keepdims=True)
        colm = jnp.min(d, axis=0, keepdims=True)
        sp = s - 1
        ip = sp // _NJ
        jp = lax.rem(sp, _NJ)

        @pl.when(jp == 0)
        def _():
            rowacc_ref[...] = rowm

        @pl.when(jp > 0)
        def _():
            rowacc_ref[...] = jnp.minimum(rowacc_ref[...], rowm)

        @pl.when(jp == _NJ - 1)
        def _():
            sum_ref[0] += jnp.sum(rowacc_ref[...])

        @pl.when(ip == 0)
        def _():
            colacc_ref[0:1, pl.ds(jp * _BJ, _BJ)] = colm

        @pl.when(ip > 0)
        def _():
            colacc_ref[0:1, pl.ds(jp * _BJ, _BJ)] = jnp.minimum(
                colacc_ref[0:1, pl.ds(jp * _BJ, _BJ)], colm)

    @pl.when(lax.rem(s, 2) == 0)
    def _():
        dbufa_ref[...] = _make_block(p_ref[...], t_ref[...])
        _reduce(dbufb_ref)

    @pl.when(lax.rem(s, 2) == 1)
    def _():
        dbufb_ref[...] = _make_block(p_ref[...], t_ref[...])
        _reduce(dbufa_ref)

    @pl.when(s == _NSTEP)
    def _():
        total = sum_ref[0] + jnp.sum(colacc_ref[...])
        out_ref[...] = jnp.full((1, 1), total / (2.0 * _N), jnp.float32)


def _pi(s):
    return (jnp.minimum(s // _NJ, _NI - 1), 0)


def _tj(s):
    return (0, lax.rem(s, _NJ))


def _chamfer(pred, target_t, interpret=False):
    return pl.pallas_call(
        _chamfer_block_kernel,
        grid=(_NSTEP + 1,),
        in_specs=[
            pl.BlockSpec((_BI, 3), _pi),
            pl.BlockSpec((3, _BJ), _tj),
        ],
        out_specs=pl.BlockSpec((1, 1), lambda s: (0, 0)),
        out_shape=jax.ShapeDtypeStruct((1, 1), jnp.float32),
        scratch_shapes=[
            pltpu.VMEM((_BI, _BJ), jnp.float32),
            pltpu.VMEM((_BI, _BJ), jnp.float32),
            pltpu.VMEM((_BI, 1), jnp.float32),
            pltpu.VMEM((1, _N), jnp.float32),
            pltpu.SMEM((1,), jnp.float32),
        ],
        interpret=interpret,
    )(pred, target_t)


@jax.jit
def kernel(pred_positions, target_positions):
    out = _chamfer(pred_positions, target_positions.T)
    return out[0, 0]
